# Initial kernel scaffold; baseline (speedup 1.0000x reference)
import jax
import jax.numpy as jnp
from jax.experimental import pallas as pl


def kernel(h, r, t, entity_embedding, relation_embedding):
    raise NotImplementedError("write your pallas kernel here")


# speedup vs baseline: 2.1487x; 2.1487x over previous
import jax
import jax.numpy as jnp
from jax import lax
from jax.experimental import pallas as pl
from jax.experimental.pallas import tpu as pltpu
from jax.experimental.pallas import tpu_sc as plsc

N_CORES = 2
N_SUBCORES = 16
N_WORKERS = N_CORES * N_SUBCORES
LANES = 16

BATCH = 16384
DIM = 128
B_W = BATCH // N_WORKERS
CHUNK = 128
N_CHUNKS = B_W // CHUNK
GROUPS = CHUNK // LANES
SLICES = DIM // LANES
TPAD = 24


def _sc_body(h_hbm, r_hbm, t_hbm, ent_hbm, rel_hbm, out_hbm,
             h_idx, r_idx, t_idx,
             hb0, rb0, tb0, hb1, rb1, tb1,
             out_v, tr, sem0, sem1):
    wid = lax.axis_index("s") * N_CORES + lax.axis_index("c")
    base = pl.multiple_of(wid * B_W, B_W)

    pltpu.sync_copy(h_hbm.at[pl.ds(base, B_W)], h_idx)
    pltpu.sync_copy(r_hbm.at[pl.ds(base, B_W)], r_idx)
    pltpu.sync_copy(t_hbm.at[pl.ds(base, B_W)], t_idx)

    bufs = ((hb0, rb0, tb0, sem0), (hb1, rb1, tb1, sem1))

    def start(c, b):
        hb, rb, tb, sem = bufs[b]
        lo = c * CHUNK
        return (
            pltpu.async_copy(ent_hbm.at[h_idx.at[pl.ds(lo, CHUNK)]], hb, sem),
            pltpu.async_copy(rel_hbm.at[r_idx.at[pl.ds(lo, CHUNK)]], rb, sem),
            pltpu.async_copy(ent_hbm.at[t_idx.at[pl.ds(lo, CHUNK)]], tb, sem),
        )

    iota = lax.iota(jnp.int32, LANES)

    def compute(c, b):
        hb, rb, tb, _ = bufs[b]

        def group(g, carry):
            row0 = pl.multiple_of(g * LANES, LANES)
            for j in range(LANES):
                row = row0 + j
                acc = jnp.zeros((LANES,), jnp.float32)
                for s in range(SLICES):
                    sl = pl.ds(s * LANES, LANES)
                    acc = acc + jnp.abs(hb[row, sl] + rb[row, sl] - tb[row, sl])
                tr[pl.ds(j * TPAD, LANES)] = acc
            tot = plsc.load_gather(tr, [iota * TPAD])
            for i in range(1, LANES):
                tot = tot + plsc.load_gather(tr, [iota * TPAD + i])
            out_v[pl.ds(c * CHUNK + row0, LANES)] = -tot
            return carry

        lax.fori_loop(0, GROUPS, group, 0)

    descs = start(0, 0)
    for c in range(N_CHUNKS):
        nxt = start(c + 1, (c + 1) % 2) if c + 1 < N_CHUNKS else None
        for d in descs:
            d.wait()
        compute(c, c % 2)
        descs = nxt

    pltpu.sync_copy(out_v, out_hbm.at[pl.ds(base, B_W)])


def _make_kernel():
    mesh = plsc.VectorSubcoreMesh(core_axis_name="c", subcore_axis_name="s",
                                  num_cores=N_CORES, num_subcores=N_SUBCORES)
    return pl.kernel(
        _sc_body,
        out_type=jax.ShapeDtypeStruct((BATCH,), jnp.float32),
        mesh=mesh,
        compiler_params=pltpu.CompilerParams(needs_layout_passes=False),
        scratch_types=[
            pltpu.VMEM((B_W,), jnp.int32),
            pltpu.VMEM((B_W,), jnp.int32),
            pltpu.VMEM((B_W,), jnp.int32),
            pltpu.VMEM((CHUNK, DIM), jnp.float32),
            pltpu.VMEM((CHUNK, DIM), jnp.float32),
            pltpu.VMEM((CHUNK, DIM), jnp.float32),
            pltpu.VMEM((CHUNK, DIM), jnp.float32),
            pltpu.VMEM((CHUNK, DIM), jnp.float32),
            pltpu.VMEM((CHUNK, DIM), jnp.float32),
            pltpu.VMEM((B_W,), jnp.float32),
            pltpu.VMEM((LANES * TPAD,), jnp.float32),
            pltpu.SemaphoreType.DMA,
            pltpu.SemaphoreType.DMA,
        ],
    )


@jax.jit
def kernel(h, r, t, entity_embedding, relation_embedding):
    fn = _make_kernel()
    return fn(h.astype(jnp.int32), r.astype(jnp.int32), t.astype(jnp.int32),
              entity_embedding, relation_embedding)
